# baseline (speedup 1.0000x reference)
import jax
import jax.numpy as jnp
from jax.experimental import pallas as pl
from jax.experimental.pallas import tpu as pltpu

N_TOK = 8192
DIM = 4096
BM = 256
BK = 256
NK = DIM // BK


def _mm_kernel(x_ref, w_ref, b_ref, o_ref, wbf_ref):
    m = pl.program_id(0)
    k = pl.program_id(1)

    @pl.when(m == 0)
    def _cast_w():
        wbf_ref[pl.ds(k * BK, BK), :] = w_ref[...].astype(jnp.bfloat16)

    xb = x_ref[...].astype(jnp.bfloat16)
    acc = jnp.dot(
        xb, wbf_ref[pl.ds(k * BK, BK), :], preferred_element_type=jnp.float32
    )

    @pl.when(k == 0)
    def _init():
        o_ref[...] = acc + b_ref[...]

    @pl.when(k != 0)
    def _accum():
        o_ref[...] += acc


def kernel(x, weight, bias):
    b2 = bias.reshape(1, DIM)
    grid = (N_TOK // BM, NK)
    return pl.pallas_call(
        _mm_kernel,
        grid=grid,
        in_specs=[
            pl.BlockSpec((BM, BK), lambda m, k: (m, k)),
            pl.BlockSpec(
                (BK, DIM), lambda m, k: (jnp.where(m == 0, k, NK - 1), 0)
            ),
            pl.BlockSpec((1, DIM), lambda m, k: (0, 0)),
        ],
        out_specs=pl.BlockSpec((BM, DIM), lambda m, k: (m, 0)),
        out_shape=jax.ShapeDtypeStruct((N_TOK, DIM), jnp.float32),
        scratch_shapes=[pltpu.VMEM((DIM, DIM), jnp.bfloat16)],
    )(x, weight, b2)


# speedup vs baseline: 1.9958x; 1.9958x over previous
import jax
import jax.numpy as jnp
from jax.experimental import pallas as pl

N_TOK = 8192
DIM = 4096
BM = 256
BCAST = 256


def _cast_kernel(w_ref, o_ref):
    o_ref[...] = w_ref[...].astype(jnp.bfloat16)


def _mm_kernel(x_ref, w_ref, b_ref, o_ref):
    xb = x_ref[...].astype(jnp.bfloat16)
    acc = jnp.dot(xb, w_ref[...], preferred_element_type=jnp.float32)
    o_ref[...] = acc + b_ref[...]


def kernel(x, weight, bias):
    wb = pl.pallas_call(
        _cast_kernel,
        grid=(DIM // BCAST,),
        in_specs=[pl.BlockSpec((BCAST, DIM), lambda k: (k, 0))],
        out_specs=pl.BlockSpec((BCAST, DIM), lambda k: (k, 0)),
        out_shape=jax.ShapeDtypeStruct((DIM, DIM), jnp.bfloat16),
    )(weight)
    b2 = bias.reshape(1, DIM)
    return pl.pallas_call(
        _mm_kernel,
        grid=(N_TOK // BM,),
        in_specs=[
            pl.BlockSpec((BM, DIM), lambda m: (m, 0)),
            pl.BlockSpec((DIM, DIM), lambda m: (0, 0)),
            pl.BlockSpec((1, DIM), lambda m: (0, 0)),
        ],
        out_specs=pl.BlockSpec((BM, DIM), lambda m: (m, 0)),
        out_shape=jax.ShapeDtypeStruct((N_TOK, DIM), jnp.float32),
    )(x, wb, b2)
